# Initial kernel scaffold; baseline (speedup 1.0000x reference)
import jax
import jax.numpy as jnp
from jax.experimental import pallas as pl


def kernel(x, edge_index, W1, b1, W2, b2):
    raise NotImplementedError("write your pallas kernel here")


# speedup vs baseline: 17.1126x; 17.1126x over previous
import functools

import jax
import jax.numpy as jnp
from jax import lax
from jax.experimental import pallas as pl
from jax.experimental.pallas import tpu as pltpu
from jax.experimental.pallas import tpu_sc as plsc

N = 10000
E = 320000
F_IN = 128
HID = 16
NCLS = 40

NC = 2
NS = 16
NW = NC * NS
CHUNK = 128
RW = 80
ER = NW * RW
EP = ER * CHUNK
NP = N + 112
RPT = NP // NS
K = 4
G = RW // K

_mesh = plsc.VectorSubcoreMesh(core_axis_name="c", subcore_axis_name="s")
_sc_params = pltpu.CompilerParams(use_tc_tiling_on_sc=False)


def _zero_rows(buf):
    def zrow(i, carry):
        buf[i, :] = jnp.zeros((16,), jnp.float32)
        return carry
    lax.fori_loop(0, buf.shape[0], zrow, 0)


def _edge_pass(table, src_hbm, dst_hbm, out_hbm, src_v, dst_v, rows_v,
               acc_sh, sem_a, sem_b, cid, sid):
    wid = sid * NC + cid
    ebase = wid * RW
    pltpu.sync_copy(src_hbm.at[pl.ds(ebase, RW)], src_v)
    pltpu.sync_copy(dst_hbm.at[pl.ds(ebase, RW)], dst_v)

    def fire(group, half, sem):
        for b in range(K):
            pltpu.async_copy(table.at[src_v.at[group * K + b]],
                             rows_v.at[half, b], sem)

    def drain_scatter(group, half, sem):
        for b in range(K):
            pltpu.make_async_copy(table.at[src_v.at[group * K + b]],
                                  rows_v.at[half, b], sem).wait()
        for b in range(K):
            pltpu.sync_copy(rows_v.at[half, b], acc_sh.at[dst_v.at[group * K + b]],
                            add=True)

    fire(0, 0, sem_a)

    def body(g2, carry):
        ga = 2 * g2
        fire(ga + 1, 1, sem_b)
        drain_scatter(ga, 0, sem_a)

        @pl.when(ga + 2 < G)
        def _():
            fire(ga + 2, 0, sem_a)

        drain_scatter(ga + 1, 1, sem_b)
        return carry

    lax.fori_loop(0, G // 2, body, 0)
    plsc.subcore_barrier()
    row0 = sid * RPT
    pltpu.sync_copy(acc_sh.at[pl.ds(row0, RPT)],
                    out_hbm.at[cid, pl.ds(row0, RPT)])


@functools.partial(
    pl.kernel,
    out_type=jax.ShapeDtypeStruct((NC, NP, HID), jnp.float32),
    mesh=_mesh,
    compiler_params=_sc_params,
    scratch_types=[
        pltpu.VMEM((RW, CHUNK), jnp.int32),
        pltpu.VMEM((RW, CHUNK), jnp.int32),
        pltpu.VMEM((2, K, CHUNK, HID), jnp.float32),
        pltpu.VMEM((RPT, HID), jnp.float32),
        pltpu.VMEM_SHARED((NP, HID), jnp.float32),
        pltpu.SemaphoreType.DMA,
        pltpu.SemaphoreType.DMA,
    ],
)
def _sc_agg1(h_hbm, src_hbm, dst_hbm, out_hbm,
             src_v, dst_v, rows_v, zbuf, acc_sh, sem_a, sem_b):
    cid = lax.axis_index("c")
    sid = lax.axis_index("s")
    row0 = sid * RPT
    _zero_rows(zbuf)
    pltpu.sync_copy(zbuf, acc_sh.at[pl.ds(row0, RPT)])
    plsc.subcore_barrier()
    _edge_pass(h_hbm, src_hbm, dst_hbm, out_hbm, src_v, dst_v, rows_v,
               acc_sh, sem_a, sem_b, cid, sid)


@functools.partial(
    pl.kernel,
    out_type=jax.ShapeDtypeStruct((NC, NP, HID), jnp.float32),
    mesh=_mesh,
    compiler_params=_sc_params,
    scratch_types=[
        pltpu.VMEM((RPT, HID), jnp.float32),
        pltpu.VMEM((RPT, HID), jnp.float32),
        pltpu.VMEM((16,), jnp.float32),
        pltpu.VMEM((RW, CHUNK), jnp.int32),
        pltpu.VMEM((RW, CHUNK), jnp.int32),
        pltpu.VMEM((2, K, CHUNK, HID), jnp.float32),
        pltpu.VMEM((RPT, HID), jnp.float32),
        pltpu.VMEM_SHARED((NP, HID), jnp.float32),
        pltpu.VMEM_SHARED((NP, HID), jnp.float32),
        pltpu.SemaphoreType.DMA,
        pltpu.SemaphoreType.DMA,
    ],
)
def _sc_agg2(p_hbm, b1_hbm, src_hbm, dst_hbm, out_hbm,
             p0_v, p1_v, b1_v, src_v, dst_v, rows_v, zbuf,
             h1_sh, acc_sh, sem_a, sem_b):
    cid = lax.axis_index("c")
    sid = lax.axis_index("s")
    row0 = sid * RPT
    pltpu.sync_copy(b1_hbm, b1_v)
    pltpu.sync_copy(p_hbm.at[0, pl.ds(row0, RPT)], p0_v)
    pltpu.sync_copy(p_hbm.at[1, pl.ds(row0, RPT)], p1_v)
    b1 = b1_v[...]

    def relu_row(i, carry):
        p0_v[i, :] = jnp.maximum(p0_v[i, :] + p1_v[i, :] + b1, 0.0)
        return carry

    lax.fori_loop(0, RPT, relu_row, 0)
    pltpu.sync_copy(p0_v, h1_sh.at[pl.ds(row0, RPT)])
    _zero_rows(zbuf)
    pltpu.sync_copy(zbuf, acc_sh.at[pl.ds(row0, RPT)])
    plsc.subcore_barrier()
    _edge_pass(h1_sh, src_hbm, dst_hbm, out_hbm, src_v, dst_v, rows_v,
               acc_sh, sem_a, sem_b, cid, sid)


def _tc_mm1(x_ref, w_ref, o_ref):
    o_ref[...] = jnp.dot(x_ref[...], w_ref[...],
                         preferred_element_type=jnp.float32,
                         precision=lax.Precision.HIGHEST)


def _tc_final(g0_ref, g1_ref, w_ref, b_ref, o_ref):
    g = g0_ref[...] + g1_ref[...]
    s = jnp.dot(g, w_ref[...], preferred_element_type=jnp.float32,
                precision=lax.Precision.HIGHEST) + b_ref[...]
    m = jnp.max(s, axis=1, keepdims=True)
    ls = jnp.log(jnp.sum(jnp.exp(s - m), axis=1, keepdims=True))
    o_ref[...] = s - m - ls


def kernel(x, edge_index, W1, b1, W2, b2):
    src = edge_index[0]
    dst = edge_index[1]
    pad = EP - E
    src_p = jnp.concatenate([src, jnp.zeros((pad,), jnp.int32)]).reshape(ER, CHUNK)
    dst_p = jnp.concatenate([dst, jnp.full((pad,), N, jnp.int32)]).reshape(ER, CHUNK)

    h = pl.pallas_call(
        _tc_mm1,
        out_shape=jax.ShapeDtypeStruct((N, HID), jnp.float32),
    )(x, W1)

    p1 = _sc_agg1(h, src_p, dst_p)
    p2 = _sc_agg2(p1, b1, src_p, dst_p)

    out = pl.pallas_call(
        _tc_final,
        out_shape=jax.ShapeDtypeStruct((N, NCLS), jnp.float32),
    )(p2[0, :N], p2[1, :N], W2, b2.reshape(1, NCLS))
    return out


# speedup vs baseline: 23.3264x; 1.3631x over previous
import functools

import jax
import jax.numpy as jnp
from jax import lax
from jax.experimental import pallas as pl
from jax.experimental.pallas import tpu as pltpu
from jax.experimental.pallas import tpu_sc as plsc

N = 10000
E = 320000
F_IN = 128
HID = 16
NCLS = 40

NC = 2
NS = 16
NW = NC * NS
CHUNK = 128
RW = 80
ER = NW * RW
EP = ER * CHUNK
NP = N + 112
RPT = NP // NS
HPT = N // NS
NBUF = 4
G = RW

_mesh = plsc.VectorSubcoreMesh(core_axis_name="c", subcore_axis_name="s")
_sc_params = pltpu.CompilerParams(use_tc_tiling_on_sc=False)


def _zero_rows(buf):
    def zrow(i, carry):
        buf[i, :] = jnp.zeros((16,), jnp.float32)
        return carry
    lax.fori_loop(0, buf.shape[0], zrow, 0)


def _edge_pass(table, src_hbm, dst_hbm, out_hbm, src_v, dst_v, rows_v,
               acc_sh, semg, sems, cid, sid):
    wid = sid * NC + cid
    ebase = wid * RW
    pltpu.sync_copy(src_hbm.at[pl.ds(ebase, RW)], src_v)
    pltpu.sync_copy(dst_hbm.at[pl.ds(ebase, RW)], dst_v)

    def fire(g, b):
        pltpu.async_copy(table.at[src_v.at[g]], rows_v.at[b], semg[b])

    def wait_gather(g, b):
        pltpu.make_async_copy(table.at[src_v.at[g]], rows_v.at[b],
                              semg[b]).wait()

    def scatter(g, b):
        pltpu.async_copy(rows_v.at[b], acc_sh.at[dst_v.at[g]], sems[b],
                         add=True)

    def wait_scatter(g, b):
        pltpu.make_async_copy(rows_v.at[b], acc_sh.at[dst_v.at[g]],
                              sems[b]).wait()

    fire(0, 0)
    fire(1, 1)

    def quad(i, carry):
        for b in range(NBUF):
            g = NBUF * i + b
            wait_gather(g, b)
            scatter(g, b)
            tb = (b + 2) % NBUF

            @pl.when(g + 2 < G)
            def _():
                @pl.when(g >= 2)
                def _():
                    wait_scatter(g - 2, tb)
                fire(g + 2, tb)
        return carry

    lax.fori_loop(0, G // NBUF, quad, 0)
    for b in range(NBUF):
        wait_scatter(G - NBUF + b, b)
    plsc.subcore_barrier()
    row0 = sid * RPT
    pltpu.sync_copy(acc_sh.at[pl.ds(row0, RPT)],
                    out_hbm.at[cid, pl.ds(row0, RPT)])


@functools.partial(
    pl.kernel,
    out_type=jax.ShapeDtypeStruct((NC, NP, HID), jnp.float32),
    mesh=_mesh,
    compiler_params=_sc_params,
    scratch_types=[
        pltpu.VMEM((RW, CHUNK), jnp.int32),
        pltpu.VMEM((RW, CHUNK), jnp.int32),
        pltpu.VMEM((NBUF, CHUNK, HID), jnp.float32),
        pltpu.VMEM((RPT, HID), jnp.float32),
        pltpu.VMEM_SHARED((NP, HID), jnp.float32),
        pltpu.VMEM_SHARED((NP, HID), jnp.float32),
        pltpu.SemaphoreType.DMA,
        [pltpu.SemaphoreType.DMA] * NBUF,
        [pltpu.SemaphoreType.DMA] * NBUF,
    ],
)
def _sc_agg1(h_hbm, src_hbm, dst_hbm, out_hbm,
             src_v, dst_v, rows_v, zbuf, tab_sh, acc_sh, semz, semg, sems):
    cid = lax.axis_index("c")
    sid = lax.axis_index("s")
    hrow = sid * HPT
    pltpu.async_copy(h_hbm.at[pl.ds(hrow, HPT)], tab_sh.at[pl.ds(hrow, HPT)],
                     semz)
    row0 = sid * RPT
    _zero_rows(zbuf)
    pltpu.sync_copy(zbuf, acc_sh.at[pl.ds(row0, RPT)])
    pltpu.make_async_copy(h_hbm.at[pl.ds(hrow, HPT)],
                          tab_sh.at[pl.ds(hrow, HPT)], semz).wait()
    plsc.subcore_barrier()
    _edge_pass(tab_sh, src_hbm, dst_hbm, out_hbm, src_v, dst_v, rows_v,
               acc_sh, semg, sems, cid, sid)


@functools.partial(
    pl.kernel,
    out_type=jax.ShapeDtypeStruct((NC, NP, HID), jnp.float32),
    mesh=_mesh,
    compiler_params=_sc_params,
    scratch_types=[
        pltpu.VMEM((RPT, HID), jnp.float32),
        pltpu.VMEM((RPT, HID), jnp.float32),
        pltpu.VMEM((16,), jnp.float32),
        pltpu.VMEM((RW, CHUNK), jnp.int32),
        pltpu.VMEM((RW, CHUNK), jnp.int32),
        pltpu.VMEM((NBUF, CHUNK, HID), jnp.float32),
        pltpu.VMEM((RPT, HID), jnp.float32),
        pltpu.VMEM_SHARED((NP, HID), jnp.float32),
        pltpu.VMEM_SHARED((NP, HID), jnp.float32),
        pltpu.SemaphoreType.DMA,
        [pltpu.SemaphoreType.DMA] * NBUF,
        [pltpu.SemaphoreType.DMA] * NBUF,
    ],
)
def _sc_agg2(p_hbm, b1_hbm, src_hbm, dst_hbm, out_hbm,
             p0_v, p1_v, b1_v, src_v, dst_v, rows_v, zbuf,
             h1_sh, acc_sh, semz, semg, sems):
    cid = lax.axis_index("c")
    sid = lax.axis_index("s")
    row0 = sid * RPT
    pltpu.async_copy(p_hbm.at[0, pl.ds(row0, RPT)], p0_v, semz)
    pltpu.async_copy(p_hbm.at[1, pl.ds(row0, RPT)], p1_v, semz)
    pltpu.sync_copy(b1_hbm, b1_v)
    _zero_rows(zbuf)
    pltpu.sync_copy(zbuf, acc_sh.at[pl.ds(row0, RPT)])
    pltpu.make_async_copy(p_hbm.at[0, pl.ds(row0, RPT)], p0_v, semz).wait()
    pltpu.make_async_copy(p_hbm.at[1, pl.ds(row0, RPT)], p1_v, semz).wait()
    b1 = b1_v[...]

    def relu_row(i, carry):
        p0_v[i, :] = jnp.maximum(p0_v[i, :] + p1_v[i, :] + b1, 0.0)
        return carry

    lax.fori_loop(0, RPT, relu_row, 0)
    pltpu.sync_copy(p0_v, h1_sh.at[pl.ds(row0, RPT)])
    plsc.subcore_barrier()
    _edge_pass(h1_sh, src_hbm, dst_hbm, out_hbm, src_v, dst_v, rows_v,
               acc_sh, semg, sems, cid, sid)


def _tc_mm1(x_ref, w_ref, o_ref):
    o_ref[...] = jnp.dot(x_ref[...], w_ref[...],
                         preferred_element_type=jnp.float32,
                         precision=lax.Precision.HIGHEST)


def _tc_final(p_ref, w_ref, b_ref, o_ref):
    g = p_ref[0, :N, :] + p_ref[1, :N, :]
    s = jnp.dot(g, w_ref[...], preferred_element_type=jnp.float32,
                precision=lax.Precision.HIGHEST) + b_ref[...]
    m = jnp.max(s, axis=1, keepdims=True)
    ls = jnp.log(jnp.sum(jnp.exp(s - m), axis=1, keepdims=True))
    o_ref[...] = s - m - ls


def kernel(x, edge_index, W1, b1, W2, b2):
    src = edge_index[0]
    dst = edge_index[1]
    pad = EP - E
    src_p = jnp.concatenate([src, jnp.zeros((pad,), jnp.int32)]).reshape(ER, CHUNK)
    dst_p = jnp.concatenate([dst, jnp.full((pad,), N, jnp.int32)]).reshape(ER, CHUNK)

    h = pl.pallas_call(
        _tc_mm1,
        out_shape=jax.ShapeDtypeStruct((N, HID), jnp.float32),
    )(x, W1)

    p1 = _sc_agg1(h, src_p, dst_p)
    p2 = _sc_agg2(p1, b1, src_p, dst_p)

    out = pl.pallas_call(
        _tc_final,
        out_shape=jax.ShapeDtypeStruct((N, NCLS), jnp.float32),
    )(p2, W2, b2.reshape(1, NCLS))
    return out


# speedup vs baseline: 26.0087x; 1.1150x over previous
import functools

import jax
import jax.numpy as jnp
from jax import lax
from jax.experimental import pallas as pl
from jax.experimental.pallas import tpu as pltpu
from jax.experimental.pallas import tpu_sc as plsc

N = 10000
E = 320000
F_IN = 128
HID = 16
NCLS = 40

NC = 2
NS = 16
NW = NC * NS
CHUNK = 128
ER = E // CHUNK
RW = ER // NW
XW = ER - RW * NW
NP = N + 112
RPT = NP // NS
HPT = N // NS
NBUF = 4
G = RW

_mesh = plsc.VectorSubcoreMesh(core_axis_name="c", subcore_axis_name="s")
_sc_params = pltpu.CompilerParams(use_tc_tiling_on_sc=False)


def _zero_rows(buf):
    def zrow(i, carry):
        buf[i, :] = jnp.zeros((16,), jnp.float32)
        return carry
    lax.fori_loop(0, buf.shape[0], zrow, 0)


def _edge_pass(table, ei_hbm, out_hbm, src_v, dst_v, rows_v,
               acc_sh, semg, sems, cid, sid):
    wid = sid * NC + cid
    base = wid * RW + jnp.minimum(wid, XW)
    extra = wid < XW

    @pl.when(extra)
    def _():
        pltpu.sync_copy(ei_hbm.at[0, pl.ds(base, RW + 1)], src_v)
        pltpu.sync_copy(ei_hbm.at[1, pl.ds(base, RW + 1)], dst_v)

    @pl.when(jnp.logical_not(extra))
    def _():
        pltpu.sync_copy(ei_hbm.at[0, pl.ds(base, RW)], src_v.at[pl.ds(0, RW)])
        pltpu.sync_copy(ei_hbm.at[1, pl.ds(base, RW)], dst_v.at[pl.ds(0, RW)])

    def fire(g, b):
        pltpu.async_copy(table.at[src_v.at[g]], rows_v.at[b], semg[b])

    def wait_gather(g, b):
        pltpu.make_async_copy(table.at[src_v.at[g]], rows_v.at[b],
                              semg[b]).wait()

    def scatter(g, b):
        pltpu.async_copy(rows_v.at[b], acc_sh.at[dst_v.at[g]], sems[b],
                         add=True)

    def wait_scatter(g, b):
        pltpu.make_async_copy(rows_v.at[b], acc_sh.at[dst_v.at[g]],
                              sems[b]).wait()

    fire(0, 0)
    fire(1, 1)

    def quad(i, carry):
        for b in range(NBUF):
            g = NBUF * i + b
            wait_gather(g, b)
            scatter(g, b)
            tb = (b + 2) % NBUF

            @pl.when(g + 2 < G)
            def _():
                @pl.when(g >= 2)
                def _():
                    wait_scatter(g - 2, tb)
                fire(g + 2, tb)
        return carry

    lax.fori_loop(0, G // NBUF, quad, 0)
    tail = G - (G // NBUF) * NBUF
    for t in range(tail):
        g = G - tail + t
        wait_gather(g, g % NBUF)
        scatter(g, g % NBUF)
    wait_scatter(G - 4, (G - 4) % NBUF)

    @pl.when(extra)
    def _():
        fire(G, (G - 4) % NBUF)
        wait_gather(G, (G - 4) % NBUF)
        scatter(G, (G - 4) % NBUF)
        wait_scatter(G, (G - 4) % NBUF)

    for g in (G - 3, G - 2, G - 1):
        wait_scatter(g, g % NBUF)
    plsc.subcore_barrier()
    row0 = sid * RPT
    pltpu.sync_copy(acc_sh.at[pl.ds(row0, RPT)],
                    out_hbm.at[cid, pl.ds(row0, RPT)])


@functools.partial(
    pl.kernel,
    out_type=jax.ShapeDtypeStruct((NC, NP, HID), jnp.float32),
    mesh=_mesh,
    compiler_params=_sc_params,
    scratch_types=[
        pltpu.VMEM((RW + 1, CHUNK), jnp.int32),
        pltpu.VMEM((RW + 1, CHUNK), jnp.int32),
        pltpu.VMEM((NBUF, CHUNK, HID), jnp.float32),
        pltpu.VMEM((RPT, HID), jnp.float32),
        pltpu.VMEM_SHARED((NP, HID), jnp.float32),
        pltpu.VMEM_SHARED((NP, HID), jnp.float32),
        pltpu.SemaphoreType.DMA,
        [pltpu.SemaphoreType.DMA] * NBUF,
        [pltpu.SemaphoreType.DMA] * NBUF,
    ],
)
def _sc_agg1(h_hbm, ei_hbm, out_hbm,
             src_v, dst_v, rows_v, zbuf, tab_sh, acc_sh, semz, semg, sems):
    cid = lax.axis_index("c")
    sid = lax.axis_index("s")
    hrow = sid * HPT
    pltpu.async_copy(h_hbm.at[pl.ds(hrow, HPT)], tab_sh.at[pl.ds(hrow, HPT)],
                     semz)
    row0 = sid * RPT
    _zero_rows(zbuf)
    pltpu.sync_copy(zbuf, acc_sh.at[pl.ds(row0, RPT)])
    pltpu.make_async_copy(h_hbm.at[pl.ds(hrow, HPT)],
                          tab_sh.at[pl.ds(hrow, HPT)], semz).wait()
    plsc.subcore_barrier()
    _edge_pass(tab_sh, ei_hbm, out_hbm, src_v, dst_v, rows_v,
               acc_sh, semg, sems, cid, sid)


@functools.partial(
    pl.kernel,
    out_type=jax.ShapeDtypeStruct((NC, NP, HID), jnp.float32),
    mesh=_mesh,
    compiler_params=_sc_params,
    scratch_types=[
        pltpu.VMEM((RPT, HID), jnp.float32),
        pltpu.VMEM((RPT, HID), jnp.float32),
        pltpu.VMEM((16,), jnp.float32),
        pltpu.VMEM((RW + 1, CHUNK), jnp.int32),
        pltpu.VMEM((RW + 1, CHUNK), jnp.int32),
        pltpu.VMEM((NBUF, CHUNK, HID), jnp.float32),
        pltpu.VMEM((RPT, HID), jnp.float32),
        pltpu.VMEM_SHARED((NP, HID), jnp.float32),
        pltpu.VMEM_SHARED((NP, HID), jnp.float32),
        pltpu.SemaphoreType.DMA,
        [pltpu.SemaphoreType.DMA] * NBUF,
        [pltpu.SemaphoreType.DMA] * NBUF,
    ],
)
def _sc_agg2(p_hbm, b1_hbm, ei_hbm, out_hbm,
             p0_v, p1_v, b1_v, src_v, dst_v, rows_v, zbuf,
             h1_sh, acc_sh, semz, semg, sems):
    cid = lax.axis_index("c")
    sid = lax.axis_index("s")
    row0 = sid * RPT
    pltpu.async_copy(p_hbm.at[0, pl.ds(row0, RPT)], p0_v, semz)
    pltpu.async_copy(p_hbm.at[1, pl.ds(row0, RPT)], p1_v, semz)
    pltpu.sync_copy(b1_hbm, b1_v)
    _zero_rows(zbuf)
    pltpu.sync_copy(zbuf, acc_sh.at[pl.ds(row0, RPT)])
    pltpu.make_async_copy(p_hbm.at[0, pl.ds(row0, RPT)], p0_v, semz).wait()
    pltpu.make_async_copy(p_hbm.at[1, pl.ds(row0, RPT)], p1_v, semz).wait()
    b1 = b1_v[...]

    def relu_row(i, carry):
        p0_v[i, :] = jnp.maximum(p0_v[i, :] + p1_v[i, :] + b1, 0.0)
        return carry

    lax.fori_loop(0, RPT, relu_row, 0)
    pltpu.sync_copy(p0_v, h1_sh.at[pl.ds(row0, RPT)])
    plsc.subcore_barrier()
    _edge_pass(h1_sh, ei_hbm, out_hbm, src_v, dst_v, rows_v,
               acc_sh, semg, sems, cid, sid)


def _tc_mm1(x_ref, w_ref, o_ref):
    o_ref[...] = jnp.dot(x_ref[...], w_ref[...],
                         preferred_element_type=jnp.float32,
                         precision=lax.Precision.HIGHEST)


def _tc_final(p_ref, w_ref, b_ref, o_ref):
    g = p_ref[0, :N, :] + p_ref[1, :N, :]
    s = jnp.dot(g, w_ref[...], preferred_element_type=jnp.float32,
                precision=lax.Precision.HIGHEST) + b_ref[...]
    m = jnp.max(s, axis=1, keepdims=True)
    ls = jnp.log(jnp.sum(jnp.exp(s - m), axis=1, keepdims=True))
    o_ref[...] = s - m - ls


def kernel(x, edge_index, W1, b1, W2, b2):
    ei3 = edge_index.reshape(2, ER, CHUNK)

    MB = 10
    h = pl.pallas_call(
        _tc_mm1,
        grid=(MB,),
        in_specs=[pl.BlockSpec((N // MB, F_IN), lambda i: (i, 0)),
                  pl.BlockSpec((F_IN, HID), lambda i: (0, 0))],
        out_specs=pl.BlockSpec((N // MB, HID), lambda i: (i, 0)),
        out_shape=jax.ShapeDtypeStruct((N, HID), jnp.float32),
    )(x, W1)

    p1 = _sc_agg1(h, ei3)
    p2 = _sc_agg2(p1, b1, ei3)

    out = pl.pallas_call(
        _tc_final,
        out_shape=jax.ShapeDtypeStruct((N, NCLS), jnp.float32),
    )(p2, W2, b2.reshape(1, NCLS))
    return out


# speedup vs baseline: 28.4934x; 1.0955x over previous
import functools

import jax
import jax.numpy as jnp
from jax import lax
from jax.experimental import pallas as pl
from jax.experimental.pallas import tpu as pltpu
from jax.experimental.pallas import tpu_sc as plsc

N = 10000
E = 320000
F_IN = 128
HID = 16
NCLS = 40

NC = 2
NS = 16
NW = NC * NS
CHUNK = 128
ER = E // CHUNK
RW = ER // NW
XW = ER - RW * NW
NP = N + 112
RPT = NP // NS
HPT = N // NS
NBUF = 4
G = RW

_mesh = plsc.VectorSubcoreMesh(core_axis_name="c", subcore_axis_name="s")
_sc_params = pltpu.CompilerParams(use_tc_tiling_on_sc=False)


def _zero_rows(buf):
    def zrow(i, carry):
        buf[i, :] = jnp.zeros((16,), jnp.float32)
        return carry
    lax.fori_loop(0, buf.shape[0], zrow, 0)


def _edge_pass(table, ei_hbm, out_hbm, src_v, dst_v, rows_v,
               acc_sh, semg, sems, cid, sid):
    wid = sid * NC + cid
    base = wid * RW + jnp.minimum(wid, XW)
    extra = wid < XW

    @pl.when(extra)
    def _():
        pltpu.sync_copy(ei_hbm.at[0, pl.ds(base, RW + 1)], src_v)
        pltpu.sync_copy(ei_hbm.at[1, pl.ds(base, RW + 1)], dst_v)

    @pl.when(jnp.logical_not(extra))
    def _():
        pltpu.sync_copy(ei_hbm.at[0, pl.ds(base, RW)], src_v.at[pl.ds(0, RW)])
        pltpu.sync_copy(ei_hbm.at[1, pl.ds(base, RW)], dst_v.at[pl.ds(0, RW)])

    def fire(g, b):
        pltpu.async_copy(table.at[src_v.at[g]], rows_v.at[b], semg[b])

    def wait_gather(g, b):
        pltpu.make_async_copy(table.at[src_v.at[g]], rows_v.at[b],
                              semg[b]).wait()

    def scatter(g, b):
        pltpu.async_copy(rows_v.at[b], acc_sh.at[dst_v.at[g]], sems[b],
                         add=True)

    def wait_scatter(g, b):
        pltpu.make_async_copy(rows_v.at[b], acc_sh.at[dst_v.at[g]],
                              sems[b]).wait()

    fire(0, 0)
    fire(1, 1)

    def quad(i, carry):
        for b in range(NBUF):
            g = NBUF * i + b
            wait_gather(g, b)
            scatter(g, b)
            tb = (b + 2) % NBUF

            @pl.when(g + 2 < G)
            def _():
                @pl.when(g >= 2)
                def _():
                    wait_scatter(g - 2, tb)
                fire(g + 2, tb)
        return carry

    lax.fori_loop(0, G // NBUF, quad, 0)
    tail = G - (G // NBUF) * NBUF
    for t in range(tail):
        g = G - tail + t
        wait_gather(g, g % NBUF)
        scatter(g, g % NBUF)
    wait_scatter(G - 4, (G - 4) % NBUF)

    @pl.when(extra)
    def _():
        fire(G, (G - 4) % NBUF)
        wait_gather(G, (G - 4) % NBUF)
        scatter(G, (G - 4) % NBUF)
        wait_scatter(G, (G - 4) % NBUF)

    for g in (G - 3, G - 2, G - 1):
        wait_scatter(g, g % NBUF)
    plsc.subcore_barrier()
    row0 = sid * RPT
    pltpu.sync_copy(acc_sh.at[pl.ds(row0, RPT)],
                    out_hbm.at[cid, pl.ds(row0, RPT)])


@functools.partial(
    pl.kernel,
    out_type=jax.ShapeDtypeStruct((NC, NP, HID), jnp.float32),
    mesh=_mesh,
    compiler_params=_sc_params,
    scratch_types=[
        pltpu.VMEM((RW + 1, CHUNK), jnp.int32),
        pltpu.VMEM((RW + 1, CHUNK), jnp.int32),
        pltpu.VMEM((NBUF, CHUNK, HID), jnp.float32),
        pltpu.VMEM((RPT, HID), jnp.float32),
        pltpu.VMEM_SHARED((NP, HID), jnp.float32),
        pltpu.VMEM_SHARED((NP, HID), jnp.float32),
        pltpu.SemaphoreType.DMA,
        [pltpu.SemaphoreType.DMA] * NBUF,
        [pltpu.SemaphoreType.DMA] * NBUF,
    ],
)
def _sc_agg1(h_hbm, ei_hbm, out_hbm,
             src_v, dst_v, rows_v, zbuf, tab_sh, acc_sh, semz, semg, sems):
    cid = lax.axis_index("c")
    sid = lax.axis_index("s")
    hrow = sid * HPT
    pltpu.async_copy(h_hbm.at[pl.ds(hrow, HPT)], tab_sh.at[pl.ds(hrow, HPT)],
                     semz)
    row0 = sid * RPT
    _zero_rows(zbuf)
    pltpu.sync_copy(zbuf, acc_sh.at[pl.ds(row0, RPT)])
    pltpu.make_async_copy(h_hbm.at[pl.ds(hrow, HPT)],
                          tab_sh.at[pl.ds(hrow, HPT)], semz).wait()
    plsc.subcore_barrier()
    _edge_pass(tab_sh, ei_hbm, out_hbm, src_v, dst_v, rows_v,
               acc_sh, semg, sems, cid, sid)


@functools.partial(
    pl.kernel,
    out_type=jax.ShapeDtypeStruct((NC, NP, HID), jnp.float32),
    mesh=_mesh,
    compiler_params=_sc_params,
    scratch_types=[
        pltpu.VMEM((RPT, HID), jnp.float32),
        pltpu.VMEM((RPT, HID), jnp.float32),
        pltpu.VMEM((16,), jnp.float32),
        pltpu.VMEM((RW + 1, CHUNK), jnp.int32),
        pltpu.VMEM((RW + 1, CHUNK), jnp.int32),
        pltpu.VMEM((NBUF, CHUNK, HID), jnp.float32),
        pltpu.VMEM((RPT, HID), jnp.float32),
        pltpu.VMEM_SHARED((NP, HID), jnp.float32),
        pltpu.VMEM_SHARED((NP, HID), jnp.float32),
        pltpu.SemaphoreType.DMA,
        [pltpu.SemaphoreType.DMA] * NBUF,
        [pltpu.SemaphoreType.DMA] * NBUF,
    ],
)
def _sc_agg2(p_hbm, b1_hbm, ei_hbm, out_hbm,
             p0_v, p1_v, b1_v, src_v, dst_v, rows_v, zbuf,
             h1_sh, acc_sh, semz, semg, sems):
    cid = lax.axis_index("c")
    sid = lax.axis_index("s")
    row0 = sid * RPT
    pltpu.async_copy(p_hbm.at[0, pl.ds(row0, RPT)], p0_v, semz)
    pltpu.async_copy(p_hbm.at[1, pl.ds(row0, RPT)], p1_v, semz)
    pltpu.sync_copy(b1_hbm, b1_v)
    _zero_rows(zbuf)
    pltpu.sync_copy(zbuf, acc_sh.at[pl.ds(row0, RPT)])
    pltpu.make_async_copy(p_hbm.at[0, pl.ds(row0, RPT)], p0_v, semz).wait()
    pltpu.make_async_copy(p_hbm.at[1, pl.ds(row0, RPT)], p1_v, semz).wait()
    b1 = b1_v[...]

    def relu_row(i, carry):
        p0_v[i, :] = jnp.maximum(p0_v[i, :] + p1_v[i, :] + b1, 0.0)
        return carry

    lax.fori_loop(0, RPT, relu_row, 0)
    pltpu.sync_copy(p0_v, h1_sh.at[pl.ds(row0, RPT)])
    plsc.subcore_barrier()
    _edge_pass(h1_sh, ei_hbm, out_hbm, src_v, dst_v, rows_v,
               acc_sh, semg, sems, cid, sid)


def _tc_mm1(x_ref, w_ref, o_ref):
    o_ref[...] = jnp.dot(x_ref[...], w_ref[...],
                         preferred_element_type=jnp.float32,
                         precision=lax.Precision.HIGHEST)


PK = 16
PR = N // PK
PC = PK * NCLS
PPR = NP * HID // (PK * HID)


def _tc_final(p_ref, w_ref, b_ref, o_ref):
    g = p_ref[0, :PR, :] + p_ref[1, :PR, :]
    s = jnp.dot(g, w_ref[...], preferred_element_type=jnp.float32,
                precision=lax.Precision.HIGHEST) + b_ref[...]
    cols = []
    for k in range(PK):
        sk = s[:, k * NCLS:(k + 1) * NCLS]
        mk = jnp.max(sk, axis=1, keepdims=True)
        ek = jnp.exp(sk - mk)
        lk = jnp.log(jnp.sum(ek, axis=1, keepdims=True))
        cols.append(sk - mk - lk)
    o_ref[...] = jnp.concatenate(cols, axis=1)


def kernel(x, edge_index, W1, b1, W2, b2):
    ei3 = edge_index.reshape(2, ER, CHUNK)

    h = pl.pallas_call(
        _tc_mm1,
        out_shape=jax.ShapeDtypeStruct((N, HID), jnp.float32),
    )(x, W1)

    p1 = _sc_agg1(h, ei3)
    p2 = _sc_agg2(p1, b1, ei3)

    w2p = jnp.kron(jnp.eye(PK, dtype=jnp.float32), W2)
    b2p = jnp.tile(b2, PK).reshape(1, PC)
    outp = pl.pallas_call(
        _tc_final,
        out_shape=jax.ShapeDtypeStruct((PR, PC), jnp.float32),
    )(p2.reshape(NC, PPR, PK * HID), w2p, b2p)
    return outp.reshape(N, NCLS)


# speedup vs baseline: 32.4482x; 1.1388x over previous
import functools

import jax
import jax.numpy as jnp
from jax import lax
from jax.experimental import pallas as pl
from jax.experimental.pallas import tpu as pltpu
from jax.experimental.pallas import tpu_sc as plsc

N = 10000
E = 320000
F_IN = 128
HID = 16
NCLS = 40

NC = 2
NS = 16
NW = NC * NS
CHUNK = 128
ER = E // CHUNK
RW = ER // NW
XW = ER - RW * NW
NP = N + 112
RPT = NP // NS
HPT = N // NS
NBUF = 6
PD = 3
G = RW

_mesh = plsc.VectorSubcoreMesh(core_axis_name="c", subcore_axis_name="s")
_sc_params = pltpu.CompilerParams(use_tc_tiling_on_sc=False)


def _zero_rows(buf):
    n = buf.shape[0]
    z = jnp.zeros((16,), jnp.float32)

    def zrow(i, carry):
        for u in range(8):
            buf[i * 8 + u, :] = z
        return carry

    lax.fori_loop(0, n // 8, zrow, 0)
    for u in range(n - (n // 8) * 8):
        buf[(n // 8) * 8 + u, :] = z


def _load_edges(ei_hbm, src_v, dst_v, wid, seme):
    base = wid * RW + jnp.minimum(wid, XW)
    extra = wid < XW

    @pl.when(extra)
    def _():
        pltpu.async_copy(ei_hbm.at[0, pl.ds(base, RW + 1)], src_v, seme)
        pltpu.async_copy(ei_hbm.at[1, pl.ds(base, RW + 1)], dst_v, seme)

    @pl.when(jnp.logical_not(extra))
    def _():
        pltpu.async_copy(ei_hbm.at[0, pl.ds(base, RW)],
                         src_v.at[pl.ds(0, RW)], seme)
        pltpu.async_copy(ei_hbm.at[1, pl.ds(base, RW)],
                         dst_v.at[pl.ds(0, RW)], seme)


def _wait_edges(ei_hbm, src_v, dst_v, wid, seme):
    extra = wid < XW

    @pl.when(extra)
    def _():
        pltpu.make_async_copy(ei_hbm.at[0, pl.ds(0, RW + 1)], src_v,
                              seme).wait()
        pltpu.make_async_copy(ei_hbm.at[1, pl.ds(0, RW + 1)], dst_v,
                              seme).wait()

    @pl.when(jnp.logical_not(extra))
    def _():
        pltpu.make_async_copy(ei_hbm.at[0, pl.ds(0, RW)],
                              src_v.at[pl.ds(0, RW)], seme).wait()
        pltpu.make_async_copy(ei_hbm.at[1, pl.ds(0, RW)],
                              dst_v.at[pl.ds(0, RW)], seme).wait()


def _edge_pass(table, out_hbm, src_v, dst_v, rows_v, acc_sh, semg, sems,
               cid, sid):
    wid = sid * NC + cid
    extra = wid < XW

    def fire(g, b):
        pltpu.async_copy(table.at[src_v.at[g]], rows_v.at[b], semg[b])

    def wait_gather(g, b):
        pltpu.make_async_copy(table.at[src_v.at[g]], rows_v.at[b],
                              semg[b]).wait()

    def scatter(g, b):
        pltpu.async_copy(rows_v.at[b], acc_sh.at[dst_v.at[g]], sems[b],
                         add=True)

    def wait_scatter(g, b):
        pltpu.make_async_copy(rows_v.at[b], acc_sh.at[dst_v.at[g]],
                              sems[b]).wait()

    for b in range(PD):
        fire(b, b)

    def ring(i, carry):
        for b in range(NBUF):
            g = NBUF * i + b
            wait_gather(g, b)
            scatter(g, b)
            tb = (b + PD) % NBUF

            @pl.when(g + PD < G)
            def _():
                @pl.when(g >= NBUF - PD)
                def _():
                    wait_scatter(g - (NBUF - PD), tb)
                fire(g + PD, tb)
        return carry

    lax.fori_loop(0, G // NBUF, ring, 0)
    wait_scatter(G - NBUF, (G - NBUF) % NBUF)

    @pl.when(extra)
    def _():
        b = G % NBUF
        fire(G, b)
        wait_gather(G, b)
        scatter(G, b)
        wait_scatter(G, b)

    for j in range(NBUF - 1):
        g = G - NBUF + 1 + j
        wait_scatter(g, g % NBUF)
    plsc.subcore_barrier()
    row0 = sid * RPT
    pltpu.sync_copy(acc_sh.at[pl.ds(row0, RPT)],
                    out_hbm.at[cid, pl.ds(row0, RPT)])


@functools.partial(
    pl.kernel,
    out_type=jax.ShapeDtypeStruct((NC, NP, HID), jnp.float32),
    mesh=_mesh,
    compiler_params=_sc_params,
    scratch_types=[
        pltpu.VMEM((RW + 1, CHUNK), jnp.int32),
        pltpu.VMEM((RW + 1, CHUNK), jnp.int32),
        pltpu.VMEM((NBUF, CHUNK, HID), jnp.float32),
        pltpu.VMEM((RPT, HID), jnp.float32),
        pltpu.VMEM_SHARED((NP, HID), jnp.float32),
        pltpu.VMEM_SHARED((NP, HID), jnp.float32),
        pltpu.SemaphoreType.DMA,
        pltpu.SemaphoreType.DMA,
        [pltpu.SemaphoreType.DMA] * NBUF,
        [pltpu.SemaphoreType.DMA] * NBUF,
    ],
)
def _sc_agg1(h_hbm, ei_hbm, out_hbm,
             src_v, dst_v, rows_v, zbuf, tab_sh, acc_sh, semz, seme,
             semg, sems):
    cid = lax.axis_index("c")
    sid = lax.axis_index("s")
    wid = sid * NC + cid
    _load_edges(ei_hbm, src_v, dst_v, wid, seme)
    hrow = sid * HPT
    pltpu.async_copy(h_hbm.at[pl.ds(hrow, HPT)], tab_sh.at[pl.ds(hrow, HPT)],
                     semz)
    row0 = sid * RPT
    _zero_rows(zbuf)
    pltpu.sync_copy(zbuf, acc_sh.at[pl.ds(row0, RPT)])
    pltpu.make_async_copy(h_hbm.at[pl.ds(hrow, HPT)],
                          tab_sh.at[pl.ds(hrow, HPT)], semz).wait()
    _wait_edges(ei_hbm, src_v, dst_v, wid, seme)
    plsc.subcore_barrier()
    _edge_pass(tab_sh, out_hbm, src_v, dst_v, rows_v, acc_sh, semg, sems,
               cid, sid)


@functools.partial(
    pl.kernel,
    out_type=jax.ShapeDtypeStruct((NC, NP, HID), jnp.float32),
    mesh=_mesh,
    compiler_params=_sc_params,
    scratch_types=[
        pltpu.VMEM((RPT, HID), jnp.float32),
        pltpu.VMEM((RPT, HID), jnp.float32),
        pltpu.VMEM((16,), jnp.float32),
        pltpu.VMEM((RW + 1, CHUNK), jnp.int32),
        pltpu.VMEM((RW + 1, CHUNK), jnp.int32),
        pltpu.VMEM((NBUF, CHUNK, HID), jnp.float32),
        pltpu.VMEM((RPT, HID), jnp.float32),
        pltpu.VMEM_SHARED((NP, HID), jnp.float32),
        pltpu.VMEM_SHARED((NP, HID), jnp.float32),
        pltpu.SemaphoreType.DMA,
        pltpu.SemaphoreType.DMA,
        [pltpu.SemaphoreType.DMA] * NBUF,
        [pltpu.SemaphoreType.DMA] * NBUF,
    ],
)
def _sc_agg2(p_hbm, b1_hbm, ei_hbm, out_hbm,
             p0_v, p1_v, b1_v, src_v, dst_v, rows_v, zbuf,
             h1_sh, acc_sh, semz, seme, semg, sems):
    cid = lax.axis_index("c")
    sid = lax.axis_index("s")
    wid = sid * NC + cid
    row0 = sid * RPT
    _load_edges(ei_hbm, src_v, dst_v, wid, seme)
    pltpu.async_copy(p_hbm.at[0, pl.ds(row0, RPT)], p0_v, semz)
    pltpu.async_copy(p_hbm.at[1, pl.ds(row0, RPT)], p1_v, semz)
    pltpu.sync_copy(b1_hbm, b1_v)
    _zero_rows(zbuf)
    pltpu.sync_copy(zbuf, acc_sh.at[pl.ds(row0, RPT)])
    pltpu.make_async_copy(p_hbm.at[0, pl.ds(row0, RPT)], p0_v, semz).wait()
    pltpu.make_async_copy(p_hbm.at[1, pl.ds(row0, RPT)], p1_v, semz).wait()
    b1 = b1_v[...]

    def relu_row(i, carry):
        for u in range(8):
            r = i * 8 + u
            p0_v[r, :] = jnp.maximum(p0_v[r, :] + p1_v[r, :] + b1, 0.0)
        return carry

    lax.fori_loop(0, RPT // 8, relu_row, 0)
    pltpu.sync_copy(p0_v, h1_sh.at[pl.ds(row0, RPT)])
    _wait_edges(ei_hbm, src_v, dst_v, wid, seme)
    plsc.subcore_barrier()
    _edge_pass(h1_sh, out_hbm, src_v, dst_v, rows_v, acc_sh, semg, sems,
               cid, sid)


def _tc_mm1(x_ref, w_ref, o_ref):
    o_ref[...] = jnp.dot(x_ref[...], w_ref[...],
                         preferred_element_type=jnp.float32,
                         precision=lax.Precision.HIGHEST)


PK = 16
PR = N // PK
PC = PK * NCLS
PPR = NP // PK


def _tc_final(p_ref, w_ref, b_ref, o_ref):
    g = p_ref[0, :PR, :] + p_ref[1, :PR, :]
    s = jnp.dot(g, w_ref[...], preferred_element_type=jnp.float32,
                precision=lax.Precision.HIGHEST) + b_ref[...]
    cols = []
    for k in range(PK):
        sk = s[:, k * NCLS:(k + 1) * NCLS]
        mk = jnp.max(sk, axis=1, keepdims=True)
        ek = jnp.exp(sk - mk)
        lk = jnp.log(jnp.sum(ek, axis=1, keepdims=True))
        cols.append(sk - mk - lk)
    o_ref[...] = jnp.concatenate(cols, axis=1)


def kernel(x, edge_index, W1, b1, W2, b2):
    ei3 = edge_index.reshape(2, ER, CHUNK)

    h = pl.pallas_call(
        _tc_mm1,
        out_shape=jax.ShapeDtypeStruct((N, HID), jnp.float32),
    )(x, W1)

    p1 = _sc_agg1(h, ei3)
    p2 = _sc_agg2(p1, b1, ei3)

    w2p = jnp.kron(jnp.eye(PK, dtype=jnp.float32), W2)
    b2p = jnp.tile(b2, PK).reshape(1, PC)
    outp = pl.pallas_call(
        _tc_final,
        out_shape=jax.ShapeDtypeStruct((PR, PC), jnp.float32),
    )(p2.reshape(NC, PPR, PK * HID), w2p, b2p)
    return outp.reshape(N, NCLS)


# speedup vs baseline: 33.6567x; 1.0372x over previous
import functools

import jax
import jax.numpy as jnp
from jax import lax
from jax.experimental import pallas as pl
from jax.experimental.pallas import tpu as pltpu
from jax.experimental.pallas import tpu_sc as plsc

N = 10000
E = 320000
F_IN = 128
HID = 16
NCLS = 40

NC = 2
NS = 16
NW = NC * NS
CHUNK = 128
ER = E // CHUNK
RW = ER // NW
XW = ER - RW * NW
NP = N + 112
RPT = NP // NS
HPT = N // NS
NBUF = 6
PD = 3
G = RW

_mesh = plsc.VectorSubcoreMesh(core_axis_name="c", subcore_axis_name="s")
_sc_params = pltpu.CompilerParams(use_tc_tiling_on_sc=False)


def _zero_rows(buf):
    n = buf.shape[0]
    z = jnp.zeros((16,), jnp.float32)

    def zrow(i, carry):
        for u in range(8):
            buf[i * 8 + u, :] = z
        return carry

    lax.fori_loop(0, n // 8, zrow, 0)
    for u in range(n - (n // 8) * 8):
        buf[(n // 8) * 8 + u, :] = z


def _load_edges(ei_hbm, src_v, dst_v, wid, seme):
    base = wid * RW + jnp.minimum(wid, XW)
    extra = wid < XW

    @pl.when(extra)
    def _():
        pltpu.async_copy(ei_hbm.at[0, pl.ds(base, RW + 1)], src_v, seme)
        pltpu.async_copy(ei_hbm.at[1, pl.ds(base, RW + 1)], dst_v, seme)

    @pl.when(jnp.logical_not(extra))
    def _():
        pltpu.async_copy(ei_hbm.at[0, pl.ds(base, RW)],
                         src_v.at[pl.ds(0, RW)], seme)
        pltpu.async_copy(ei_hbm.at[1, pl.ds(base, RW)],
                         dst_v.at[pl.ds(0, RW)], seme)


def _wait_edges(ei_hbm, src_v, dst_v, wid, seme):
    extra = wid < XW

    @pl.when(extra)
    def _():
        pltpu.make_async_copy(ei_hbm.at[0, pl.ds(0, RW + 1)], src_v,
                              seme).wait()
        pltpu.make_async_copy(ei_hbm.at[1, pl.ds(0, RW + 1)], dst_v,
                              seme).wait()

    @pl.when(jnp.logical_not(extra))
    def _():
        pltpu.make_async_copy(ei_hbm.at[0, pl.ds(0, RW)],
                              src_v.at[pl.ds(0, RW)], seme).wait()
        pltpu.make_async_copy(ei_hbm.at[1, pl.ds(0, RW)],
                              dst_v.at[pl.ds(0, RW)], seme).wait()


def _edge_pass(table, out_hbm, src_v, dst_v, rows_v, acc_sh, semg, sems,
               cid, sid):
    wid = sid * NC + cid
    extra = wid < XW

    def fire(g, b):
        pltpu.async_copy(table.at[src_v.at[g]], rows_v.at[b], semg[b])

    def wait_gather(g, b):
        pltpu.make_async_copy(table.at[src_v.at[g]], rows_v.at[b],
                              semg[b]).wait()

    def scatter(g, b):
        pltpu.async_copy(rows_v.at[b], acc_sh.at[dst_v.at[g]], sems[b],
                         add=True)

    def wait_scatter(g, b):
        pltpu.make_async_copy(rows_v.at[b], acc_sh.at[dst_v.at[g]],
                              sems[b]).wait()

    for b in range(PD):
        fire(b, b)

    def ring(i, carry):
        for b in range(NBUF):
            g = NBUF * i + b
            wait_gather(g, b)
            scatter(g, b)
            tb = (b + PD) % NBUF

            @pl.when(g + PD < G)
            def _():
                @pl.when(g >= NBUF - PD)
                def _():
                    wait_scatter(g - (NBUF - PD), tb)
                fire(g + PD, tb)
        return carry

    lax.fori_loop(0, G // NBUF, ring, 0)
    wait_scatter(G - NBUF, (G - NBUF) % NBUF)

    @pl.when(extra)
    def _():
        b = G % NBUF
        fire(G, b)
        wait_gather(G, b)
        scatter(G, b)
        wait_scatter(G, b)

    for j in range(NBUF - 1):
        g = G - NBUF + 1 + j
        wait_scatter(g, g % NBUF)
    plsc.subcore_barrier()
    row0 = sid * RPT
    pltpu.sync_copy(acc_sh.at[pl.ds(row0, RPT)],
                    out_hbm.at[cid, pl.ds(row0, RPT)])


@functools.partial(
    pl.kernel,
    out_type=jax.ShapeDtypeStruct((NC, NP, HID), jnp.float32),
    mesh=_mesh,
    compiler_params=_sc_params,
    scratch_types=[
        pltpu.VMEM((RW + 1, CHUNK), jnp.int32),
        pltpu.VMEM((RW + 1, CHUNK), jnp.int32),
        pltpu.VMEM((NBUF, CHUNK, HID), jnp.float32),
        pltpu.VMEM((RPT, HID), jnp.float32),
        pltpu.VMEM_SHARED((NP, HID), jnp.float32),
        pltpu.VMEM_SHARED((NP, HID), jnp.float32),
        pltpu.SemaphoreType.DMA,
        pltpu.SemaphoreType.DMA,
        [pltpu.SemaphoreType.DMA] * NBUF,
        [pltpu.SemaphoreType.DMA] * NBUF,
    ],
)
def _sc_agg1(h_hbm, ei_hbm, out_hbm,
             src_v, dst_v, rows_v, zbuf, tab_sh, acc_sh, semz, seme,
             semg, sems):
    cid = lax.axis_index("c")
    sid = lax.axis_index("s")
    wid = sid * NC + cid
    _load_edges(ei_hbm, src_v, dst_v, wid, seme)
    hrow = sid * HPT
    pltpu.async_copy(h_hbm.at[pl.ds(hrow, HPT)], tab_sh.at[pl.ds(hrow, HPT)],
                     semz)
    row0 = sid * RPT
    _zero_rows(zbuf)
    pltpu.sync_copy(zbuf, acc_sh.at[pl.ds(row0, RPT)])
    pltpu.make_async_copy(h_hbm.at[pl.ds(hrow, HPT)],
                          tab_sh.at[pl.ds(hrow, HPT)], semz).wait()
    _wait_edges(ei_hbm, src_v, dst_v, wid, seme)
    plsc.subcore_barrier()
    _edge_pass(tab_sh, out_hbm, src_v, dst_v, rows_v, acc_sh, semg, sems,
               cid, sid)


@functools.partial(
    pl.kernel,
    out_type=jax.ShapeDtypeStruct((NC, NP, HID), jnp.float32),
    mesh=_mesh,
    compiler_params=_sc_params,
    scratch_types=[
        pltpu.VMEM((RPT, HID), jnp.float32),
        pltpu.VMEM((RPT, HID), jnp.float32),
        pltpu.VMEM((16,), jnp.float32),
        pltpu.VMEM((RW + 1, CHUNK), jnp.int32),
        pltpu.VMEM((RW + 1, CHUNK), jnp.int32),
        pltpu.VMEM((NBUF, CHUNK, HID), jnp.float32),
        pltpu.VMEM((RPT, HID), jnp.float32),
        pltpu.VMEM_SHARED((NP, HID), jnp.float32),
        pltpu.VMEM_SHARED((NP, HID), jnp.float32),
        pltpu.SemaphoreType.DMA,
        pltpu.SemaphoreType.DMA,
        [pltpu.SemaphoreType.DMA] * NBUF,
        [pltpu.SemaphoreType.DMA] * NBUF,
    ],
)
def _sc_agg2(p_hbm, b1_hbm, ei_hbm, out_hbm,
             p0_v, p1_v, b1_v, src_v, dst_v, rows_v, zbuf,
             h1_sh, acc_sh, semz, seme, semg, sems):
    cid = lax.axis_index("c")
    sid = lax.axis_index("s")
    wid = sid * NC + cid
    row0 = sid * RPT
    _load_edges(ei_hbm, src_v, dst_v, wid, seme)
    pltpu.async_copy(p_hbm.at[0, pl.ds(row0, RPT)], p0_v, semz)
    pltpu.async_copy(p_hbm.at[1, pl.ds(row0, RPT)], p1_v, semz)
    pltpu.sync_copy(b1_hbm, b1_v)
    _zero_rows(zbuf)
    pltpu.sync_copy(zbuf, acc_sh.at[pl.ds(row0, RPT)])
    pltpu.make_async_copy(p_hbm.at[0, pl.ds(row0, RPT)], p0_v, semz).wait()
    pltpu.make_async_copy(p_hbm.at[1, pl.ds(row0, RPT)], p1_v, semz).wait()
    b1 = b1_v[...]

    def relu_row(i, carry):
        for u in range(8):
            r = i * 8 + u
            p0_v[r, :] = jnp.maximum(p0_v[r, :] + p1_v[r, :] + b1, 0.0)
        return carry

    lax.fori_loop(0, RPT // 8, relu_row, 0)
    pltpu.sync_copy(p0_v, h1_sh.at[pl.ds(row0, RPT)])
    _wait_edges(ei_hbm, src_v, dst_v, wid, seme)
    plsc.subcore_barrier()
    _edge_pass(h1_sh, out_hbm, src_v, dst_v, rows_v, acc_sh, semg, sems,
               cid, sid)


def _tc_mm1(x_ref, w_ref, o_ref):
    o_ref[...] = jnp.dot(x_ref[...], w_ref[...],
                         preferred_element_type=jnp.float32)


PK = 16
PR = N // PK
PC = PK * NCLS
PPR = NP // PK


def _tc_final(p_ref, w_ref, b_ref, o_ref):
    g = p_ref[0, :PR, :] + p_ref[1, :PR, :]
    s = jnp.dot(g, w_ref[...], preferred_element_type=jnp.float32,
                precision=lax.Precision.HIGHEST) + b_ref[...]
    cols = []
    for k in range(PK):
        sk = s[:, k * NCLS:(k + 1) * NCLS]
        mk = jnp.max(sk, axis=1, keepdims=True)
        ek = jnp.exp(sk - mk)
        lk = jnp.log(jnp.sum(ek, axis=1, keepdims=True))
        cols.append(sk - mk - lk)
    o_ref[...] = jnp.concatenate(cols, axis=1)


def kernel(x, edge_index, W1, b1, W2, b2):
    ei3 = edge_index.reshape(2, ER, CHUNK)

    h = pl.pallas_call(
        _tc_mm1,
        out_shape=jax.ShapeDtypeStruct((N, HID), jnp.float32),
    )(x, W1)

    p1 = _sc_agg1(h, ei3)
    p2 = _sc_agg2(p1, b1, ei3)

    w2p = jnp.kron(jnp.eye(PK, dtype=jnp.float32), W2)
    b2p = jnp.tile(b2, PK).reshape(1, PC)
    outp = pl.pallas_call(
        _tc_final,
        out_shape=jax.ShapeDtypeStruct((PR, PC), jnp.float32),
    )(p2.reshape(NC, PPR, PK * HID), w2p, b2p)
    return outp.reshape(N, NCLS)


# speedup vs baseline: 34.2892x; 1.0188x over previous
import functools

import jax
import jax.numpy as jnp
from jax import lax
from jax.experimental import pallas as pl
from jax.experimental.pallas import tpu as pltpu
from jax.experimental.pallas import tpu_sc as plsc

N = 10000
E = 320000
F_IN = 128
HID = 16
NCLS = 40

NC = 2
NS = 16
NW = NC * NS
CHUNK = 128
ER = E // CHUNK
RW = ER // NW
XW = ER - RW * NW
NP = N + 112
RPT = NP // NS
HPT = N // NS
NBUF = 6
PD = 3
G = RW

_mesh = plsc.VectorSubcoreMesh(core_axis_name="c", subcore_axis_name="s")
_sc_params = pltpu.CompilerParams(use_tc_tiling_on_sc=False)


def _zero_rows(buf):
    n = buf.shape[0]
    z = jnp.zeros((16,), jnp.float32)

    def zrow(i, carry):
        for u in range(8):
            buf[i * 8 + u, :] = z
        return carry

    lax.fori_loop(0, n // 8, zrow, 0)
    for u in range(n - (n // 8) * 8):
        buf[(n // 8) * 8 + u, :] = z


def _load_edges(ei_hbm, src_v, dst_v, wid, seme):
    base = wid * RW + jnp.minimum(wid, XW)
    extra = wid < XW

    @pl.when(extra)
    def _():
        pltpu.async_copy(ei_hbm.at[0, pl.ds(base, RW + 1)], src_v, seme)
        pltpu.async_copy(ei_hbm.at[1, pl.ds(base, RW + 1)], dst_v, seme)

    @pl.when(jnp.logical_not(extra))
    def _():
        pltpu.async_copy(ei_hbm.at[0, pl.ds(base, RW)],
                         src_v.at[pl.ds(0, RW)], seme)
        pltpu.async_copy(ei_hbm.at[1, pl.ds(base, RW)],
                         dst_v.at[pl.ds(0, RW)], seme)


def _wait_edges(ei_hbm, src_v, dst_v, wid, seme):
    extra = wid < XW

    @pl.when(extra)
    def _():
        pltpu.make_async_copy(ei_hbm.at[0, pl.ds(0, RW + 1)], src_v,
                              seme).wait()
        pltpu.make_async_copy(ei_hbm.at[1, pl.ds(0, RW + 1)], dst_v,
                              seme).wait()

    @pl.when(jnp.logical_not(extra))
    def _():
        pltpu.make_async_copy(ei_hbm.at[0, pl.ds(0, RW)],
                              src_v.at[pl.ds(0, RW)], seme).wait()
        pltpu.make_async_copy(ei_hbm.at[1, pl.ds(0, RW)],
                              dst_v.at[pl.ds(0, RW)], seme).wait()


def _edge_pass(table, out_hbm, src_v, dst_v, rows_v, acc_sh, semg, sems,
               cid, sid):
    wid = sid * NC + cid
    extra = wid < XW

    def fire(g, b):
        pltpu.async_copy(table.at[src_v.at[g]], rows_v.at[b], semg[b])

    def wait_gather(g, b):
        pltpu.make_async_copy(table.at[src_v.at[g]], rows_v.at[b],
                              semg[b]).wait()

    def scatter(g, b):
        pltpu.async_copy(rows_v.at[b], acc_sh.at[dst_v.at[g]], sems[b],
                         add=True)

    def wait_scatter(g, b):
        pltpu.make_async_copy(rows_v.at[b], acc_sh.at[dst_v.at[g]],
                              sems[b]).wait()

    for b in range(PD):
        fire(b, b)

    def ring(i, carry):
        for b in range(NBUF):
            g = NBUF * i + b
            wait_gather(g, b)
            scatter(g, b)
            tb = (b + PD) % NBUF

            @pl.when(g + PD < G)
            def _():
                @pl.when(g >= NBUF - PD)
                def _():
                    wait_scatter(g - (NBUF - PD), tb)
                fire(g + PD, tb)
        return carry

    lax.fori_loop(0, G // NBUF, ring, 0)
    wait_scatter(G - NBUF, (G - NBUF) % NBUF)

    @pl.when(extra)
    def _():
        b = G % NBUF
        fire(G, b)
        wait_gather(G, b)
        scatter(G, b)
        wait_scatter(G, b)

    for j in range(NBUF - 1):
        g = G - NBUF + 1 + j
        wait_scatter(g, g % NBUF)
    plsc.subcore_barrier()
    row0 = sid * RPT
    pltpu.sync_copy(acc_sh.at[pl.ds(row0, RPT)],
                    out_hbm.at[cid, pl.ds(row0, RPT)])


@functools.partial(
    pl.kernel,
    out_type=jax.ShapeDtypeStruct((NC, NP, HID), jnp.float32),
    mesh=_mesh,
    compiler_params=_sc_params,
    scratch_types=[
        pltpu.VMEM((RW + 1, CHUNK), jnp.int32),
        pltpu.VMEM((RW + 1, CHUNK), jnp.int32),
        pltpu.VMEM((NBUF, CHUNK, HID), jnp.float32),
        pltpu.VMEM((RPT, HID), jnp.float32),
        pltpu.VMEM_SHARED((NP, HID), jnp.float32),
        pltpu.VMEM_SHARED((NP, HID), jnp.float32),
        pltpu.SemaphoreType.DMA,
        pltpu.SemaphoreType.DMA,
        [pltpu.SemaphoreType.DMA] * NBUF,
        [pltpu.SemaphoreType.DMA] * NBUF,
    ],
)
def _sc_agg1(h_hbm, ei_hbm, out_hbm,
             src_v, dst_v, rows_v, zbuf, tab_sh, acc_sh, semz, seme,
             semg, sems):
    cid = lax.axis_index("c")
    sid = lax.axis_index("s")
    wid = sid * NC + cid
    _load_edges(ei_hbm, src_v, dst_v, wid, seme)
    hrow = sid * HPT
    pltpu.async_copy(h_hbm.at[pl.ds(hrow, HPT)], tab_sh.at[pl.ds(hrow, HPT)],
                     semz)
    row0 = sid * RPT
    _zero_rows(zbuf)
    pltpu.sync_copy(zbuf, acc_sh.at[pl.ds(row0, RPT)])
    pltpu.make_async_copy(h_hbm.at[pl.ds(hrow, HPT)],
                          tab_sh.at[pl.ds(hrow, HPT)], semz).wait()
    _wait_edges(ei_hbm, src_v, dst_v, wid, seme)
    plsc.subcore_barrier()
    _edge_pass(tab_sh, out_hbm, src_v, dst_v, rows_v, acc_sh, semg, sems,
               cid, sid)


@functools.partial(
    pl.kernel,
    out_type=jax.ShapeDtypeStruct((NC, NP, HID), jnp.float32),
    mesh=_mesh,
    compiler_params=_sc_params,
    scratch_types=[
        pltpu.VMEM((RPT, HID), jnp.float32),
        pltpu.VMEM((RPT, HID), jnp.float32),
        pltpu.VMEM((16,), jnp.float32),
        pltpu.VMEM((RW + 1, CHUNK), jnp.int32),
        pltpu.VMEM((RW + 1, CHUNK), jnp.int32),
        pltpu.VMEM((NBUF, CHUNK, HID), jnp.float32),
        pltpu.VMEM((RPT, HID), jnp.float32),
        pltpu.VMEM_SHARED((NP, HID), jnp.float32),
        pltpu.VMEM_SHARED((NP, HID), jnp.float32),
        pltpu.SemaphoreType.DMA,
        pltpu.SemaphoreType.DMA,
        [pltpu.SemaphoreType.DMA] * NBUF,
        [pltpu.SemaphoreType.DMA] * NBUF,
    ],
)
def _sc_agg2(p_hbm, b1_hbm, ei_hbm, out_hbm,
             p0_v, p1_v, b1_v, src_v, dst_v, rows_v, zbuf,
             h1_sh, acc_sh, semz, seme, semg, sems):
    cid = lax.axis_index("c")
    sid = lax.axis_index("s")
    wid = sid * NC + cid
    row0 = sid * RPT
    _load_edges(ei_hbm, src_v, dst_v, wid, seme)
    pltpu.async_copy(p_hbm.at[0, pl.ds(row0, RPT)], p0_v, semz)
    pltpu.async_copy(p_hbm.at[1, pl.ds(row0, RPT)], p1_v, semz)
    pltpu.sync_copy(b1_hbm, b1_v)
    _zero_rows(zbuf)
    pltpu.sync_copy(zbuf, acc_sh.at[pl.ds(row0, RPT)])
    pltpu.make_async_copy(p_hbm.at[0, pl.ds(row0, RPT)], p0_v, semz).wait()
    pltpu.make_async_copy(p_hbm.at[1, pl.ds(row0, RPT)], p1_v, semz).wait()
    b1 = b1_v[...]

    def relu_row(i, carry):
        for u in range(8):
            r = i * 8 + u
            p0_v[r, :] = jnp.maximum(p0_v[r, :] + p1_v[r, :] + b1, 0.0)
        return carry

    lax.fori_loop(0, RPT // 8, relu_row, 0)
    pltpu.sync_copy(p0_v, h1_sh.at[pl.ds(row0, RPT)])
    _wait_edges(ei_hbm, src_v, dst_v, wid, seme)
    plsc.subcore_barrier()
    _edge_pass(h1_sh, out_hbm, src_v, dst_v, rows_v, acc_sh, semg, sems,
               cid, sid)


def _tc_mm1(x_ref, w_ref, o_ref):
    o_ref[...] = jnp.dot(x_ref[...], w_ref[...],
                         preferred_element_type=jnp.float32)


PK = 16
PR = N // PK
PC = PK * NCLS
PPR = NP // PK


def _tc_final(p_ref, w_ref, b_ref, o_ref):
    g = p_ref[0, :PR, :] + p_ref[1, :PR, :]
    s = jnp.dot(g, w_ref[...], preferred_element_type=jnp.float32) + b_ref[...]
    cols = []
    for k in range(PK):
        sk = s[:, k * NCLS:(k + 1) * NCLS]
        mk = jnp.max(sk, axis=1, keepdims=True)
        ek = jnp.exp(sk - mk)
        lk = jnp.log(jnp.sum(ek, axis=1, keepdims=True))
        cols.append(sk - mk - lk)
    o_ref[...] = jnp.concatenate(cols, axis=1)


def kernel(x, edge_index, W1, b1, W2, b2):
    ei3 = edge_index.reshape(2, ER, CHUNK)

    h = pl.pallas_call(
        _tc_mm1,
        out_shape=jax.ShapeDtypeStruct((N, HID), jnp.float32),
    )(x, W1)

    p1 = _sc_agg1(h, ei3)
    p2 = _sc_agg2(p1, b1, ei3)

    w2p = jnp.kron(jnp.eye(PK, dtype=jnp.float32), W2)
    b2p = jnp.tile(b2, PK).reshape(1, PC)
    outp = pl.pallas_call(
        _tc_final,
        out_shape=jax.ShapeDtypeStruct((PR, PC), jnp.float32),
    )(p2.reshape(NC, PPR, PK * HID), w2p, b2p)
    return outp.reshape(N, NCLS)


# speedup vs baseline: 34.5723x; 1.0083x over previous
import functools

import jax
import jax.numpy as jnp
from jax import lax
from jax.experimental import pallas as pl
from jax.experimental.pallas import tpu as pltpu
from jax.experimental.pallas import tpu_sc as plsc

N = 10000
E = 320000
F_IN = 128
HID = 16
NCLS = 40

NC = 2
NS = 16
NW = NC * NS
CHUNK = 128
ER = E // CHUNK
RW = ER // NW
XW = ER - RW * NW
NP = N + 112
RPT = NP // NS
HPT = N // NS
NBUF = 6
PD = 4
G = RW

_mesh = plsc.VectorSubcoreMesh(core_axis_name="c", subcore_axis_name="s")
_sc_params = pltpu.CompilerParams(use_tc_tiling_on_sc=False)


def _zero_rows(buf):
    n = buf.shape[0]
    z = jnp.zeros((16,), jnp.float32)

    def zrow(i, carry):
        for u in range(8):
            buf[i * 8 + u, :] = z
        return carry

    lax.fori_loop(0, n // 8, zrow, 0)
    for u in range(n - (n // 8) * 8):
        buf[(n // 8) * 8 + u, :] = z


def _load_edges(ei_hbm, src_v, dst_v, wid, seme):
    base = wid * RW + jnp.minimum(wid, XW)
    extra = wid < XW

    @pl.when(extra)
    def _():
        pltpu.async_copy(ei_hbm.at[0, pl.ds(base, RW + 1)], src_v, seme)
        pltpu.async_copy(ei_hbm.at[1, pl.ds(base, RW + 1)], dst_v, seme)

    @pl.when(jnp.logical_not(extra))
    def _():
        pltpu.async_copy(ei_hbm.at[0, pl.ds(base, RW)],
                         src_v.at[pl.ds(0, RW)], seme)
        pltpu.async_copy(ei_hbm.at[1, pl.ds(base, RW)],
                         dst_v.at[pl.ds(0, RW)], seme)


def _wait_edges(ei_hbm, src_v, dst_v, wid, seme):
    extra = wid < XW

    @pl.when(extra)
    def _():
        pltpu.make_async_copy(ei_hbm.at[0, pl.ds(0, RW + 1)], src_v,
                              seme).wait()
        pltpu.make_async_copy(ei_hbm.at[1, pl.ds(0, RW + 1)], dst_v,
                              seme).wait()

    @pl.when(jnp.logical_not(extra))
    def _():
        pltpu.make_async_copy(ei_hbm.at[0, pl.ds(0, RW)],
                              src_v.at[pl.ds(0, RW)], seme).wait()
        pltpu.make_async_copy(ei_hbm.at[1, pl.ds(0, RW)],
                              dst_v.at[pl.ds(0, RW)], seme).wait()


def _edge_pass(table, out_hbm, src_v, dst_v, rows_v, acc_sh, semg, sems,
               cid, sid):
    wid = sid * NC + cid
    extra = wid < XW

    def fire(g, b):
        pltpu.async_copy(table.at[src_v.at[g]], rows_v.at[b], semg[b])

    def wait_gather(g, b):
        pltpu.make_async_copy(table.at[src_v.at[g]], rows_v.at[b],
                              semg[b]).wait()

    def scatter(g, b):
        pltpu.async_copy(rows_v.at[b], acc_sh.at[dst_v.at[g]], sems[b],
                         add=True)

    def wait_scatter(g, b):
        pltpu.make_async_copy(rows_v.at[b], acc_sh.at[dst_v.at[g]],
                              sems[b]).wait()

    for b in range(PD):
        fire(b, b)

    def ring(i, carry):
        for b in range(NBUF):
            g = NBUF * i + b
            wait_gather(g, b)
            scatter(g, b)
            tb = (b + PD) % NBUF

            @pl.when(g + PD < G)
            def _():
                @pl.when(g >= NBUF - PD)
                def _():
                    wait_scatter(g - (NBUF - PD), tb)
                fire(g + PD, tb)
        return carry

    lax.fori_loop(0, G // NBUF, ring, 0)
    wait_scatter(G - NBUF, (G - NBUF) % NBUF)

    @pl.when(extra)
    def _():
        b = G % NBUF
        fire(G, b)
        wait_gather(G, b)
        scatter(G, b)
        wait_scatter(G, b)

    for j in range(NBUF - 1):
        g = G - NBUF + 1 + j
        wait_scatter(g, g % NBUF)
    plsc.subcore_barrier()
    row0 = sid * RPT
    pltpu.sync_copy(acc_sh.at[pl.ds(row0, RPT)],
                    out_hbm.at[cid, pl.ds(row0, RPT)])


@functools.partial(
    pl.kernel,
    out_type=jax.ShapeDtypeStruct((NC, NP, HID), jnp.float32),
    mesh=_mesh,
    compiler_params=_sc_params,
    scratch_types=[
        pltpu.VMEM((RW + 1, CHUNK), jnp.int32),
        pltpu.VMEM((RW + 1, CHUNK), jnp.int32),
        pltpu.VMEM((NBUF, CHUNK, HID), jnp.float32),
        pltpu.VMEM((RPT, HID), jnp.float32),
        pltpu.VMEM_SHARED((NP, HID), jnp.float32),
        pltpu.VMEM_SHARED((NP, HID), jnp.float32),
        pltpu.SemaphoreType.DMA,
        pltpu.SemaphoreType.DMA,
        [pltpu.SemaphoreType.DMA] * NBUF,
        [pltpu.SemaphoreType.DMA] * NBUF,
    ],
)
def _sc_agg1(h_hbm, ei_hbm, out_hbm,
             src_v, dst_v, rows_v, zbuf, tab_sh, acc_sh, semz, seme,
             semg, sems):
    cid = lax.axis_index("c")
    sid = lax.axis_index("s")
    wid = sid * NC + cid
    _load_edges(ei_hbm, src_v, dst_v, wid, seme)
    hrow = sid * HPT
    pltpu.async_copy(h_hbm.at[pl.ds(hrow, HPT)], tab_sh.at[pl.ds(hrow, HPT)],
                     semz)
    row0 = sid * RPT
    _zero_rows(zbuf)
    pltpu.sync_copy(zbuf, acc_sh.at[pl.ds(row0, RPT)])
    pltpu.make_async_copy(h_hbm.at[pl.ds(hrow, HPT)],
                          tab_sh.at[pl.ds(hrow, HPT)], semz).wait()
    _wait_edges(ei_hbm, src_v, dst_v, wid, seme)
    plsc.subcore_barrier()
    _edge_pass(tab_sh, out_hbm, src_v, dst_v, rows_v, acc_sh, semg, sems,
               cid, sid)


@functools.partial(
    pl.kernel,
    out_type=jax.ShapeDtypeStruct((NC, NP, HID), jnp.float32),
    mesh=_mesh,
    compiler_params=_sc_params,
    scratch_types=[
        pltpu.VMEM((RPT, HID), jnp.float32),
        pltpu.VMEM((RPT, HID), jnp.float32),
        pltpu.VMEM((16,), jnp.float32),
        pltpu.VMEM((RW + 1, CHUNK), jnp.int32),
        pltpu.VMEM((RW + 1, CHUNK), jnp.int32),
        pltpu.VMEM((NBUF, CHUNK, HID), jnp.float32),
        pltpu.VMEM((RPT, HID), jnp.float32),
        pltpu.VMEM_SHARED((NP, HID), jnp.float32),
        pltpu.VMEM_SHARED((NP, HID), jnp.float32),
        pltpu.SemaphoreType.DMA,
        pltpu.SemaphoreType.DMA,
        [pltpu.SemaphoreType.DMA] * NBUF,
        [pltpu.SemaphoreType.DMA] * NBUF,
    ],
)
def _sc_agg2(p_hbm, b1_hbm, ei_hbm, out_hbm,
             p0_v, p1_v, b1_v, src_v, dst_v, rows_v, zbuf,
             h1_sh, acc_sh, semz, seme, semg, sems):
    cid = lax.axis_index("c")
    sid = lax.axis_index("s")
    wid = sid * NC + cid
    row0 = sid * RPT
    _load_edges(ei_hbm, src_v, dst_v, wid, seme)
    pltpu.async_copy(p_hbm.at[0, pl.ds(row0, RPT)], p0_v, semz)
    pltpu.async_copy(p_hbm.at[1, pl.ds(row0, RPT)], p1_v, semz)
    pltpu.sync_copy(b1_hbm, b1_v)
    _zero_rows(zbuf)
    pltpu.sync_copy(zbuf, acc_sh.at[pl.ds(row0, RPT)])
    pltpu.make_async_copy(p_hbm.at[0, pl.ds(row0, RPT)], p0_v, semz).wait()
    pltpu.make_async_copy(p_hbm.at[1, pl.ds(row0, RPT)], p1_v, semz).wait()
    b1 = b1_v[...]

    def relu_row(i, carry):
        for u in range(8):
            r = i * 8 + u
            p0_v[r, :] = jnp.maximum(p0_v[r, :] + p1_v[r, :] + b1, 0.0)
        return carry

    lax.fori_loop(0, RPT // 8, relu_row, 0)
    pltpu.sync_copy(p0_v, h1_sh.at[pl.ds(row0, RPT)])
    _wait_edges(ei_hbm, src_v, dst_v, wid, seme)
    plsc.subcore_barrier()
    _edge_pass(h1_sh, out_hbm, src_v, dst_v, rows_v, acc_sh, semg, sems,
               cid, sid)


def _tc_mm1(x_ref, w_ref, o_ref):
    o_ref[...] = jnp.dot(x_ref[...], w_ref[...],
                         preferred_element_type=jnp.float32)


PK = 16
PR = N // PK
PC = PK * NCLS
PPR = NP // PK


def _tc_final(p_ref, w_ref, b_ref, o_ref):
    g = p_ref[0, :PR, :] + p_ref[1, :PR, :]
    s = jnp.dot(g, w_ref[...], preferred_element_type=jnp.float32) + b_ref[...]
    cols = []
    for k in range(PK):
        sk = s[:, k * NCLS:(k + 1) * NCLS]
        mk = jnp.max(sk, axis=1, keepdims=True)
        ek = jnp.exp(sk - mk)
        lk = jnp.log(jnp.sum(ek, axis=1, keepdims=True))
        cols.append(sk - mk - lk)
    o_ref[...] = jnp.concatenate(cols, axis=1)


def kernel(x, edge_index, W1, b1, W2, b2):
    ei3 = edge_index.reshape(2, ER, CHUNK)

    h = pl.pallas_call(
        _tc_mm1,
        out_shape=jax.ShapeDtypeStruct((N, HID), jnp.float32),
    )(x, W1)

    p1 = _sc_agg1(h, ei3)
    p2 = _sc_agg2(p1, b1, ei3)

    w2p = jnp.kron(jnp.eye(PK, dtype=jnp.float32), W2)
    b2p = jnp.tile(b2, PK).reshape(1, PC)
    outp = pl.pallas_call(
        _tc_final,
        out_shape=jax.ShapeDtypeStruct((PR, PC), jnp.float32),
    )(p2.reshape(NC, PPR, PK * HID), w2p, b2p)
    return outp.reshape(N, NCLS)
